# baseline (speedup 1.0000x reference)
import jax
import jax.numpy as jnp
from jax import lax
from jax.experimental import pallas as pl
from jax.experimental.pallas import tpu as pltpu
from jax.experimental.pallas import tpu_sc as plsc

N = 10000
E = 160000
H = 256
NUMF = 16
CATN = 4
CAT_SIZES = (8, 12, 16, 24)
SPAD = 32
CAT_DIM = 16
EPS = 1e-5

BR = 5000

NC = 2
NS = 16
NW = NC * NS
EC = 128
NCHUNK = -(-E // EC)
WPW = -(-NCHUNK // NW)
NCHUNKP = WPW * NW
EPAD = NCHUNKP * EC
CPT = -(-NCHUNK // NS)
NPAD = 10240
RPT = NPAD // NS
CB = 128
NBUF = 3


def _erf(x):
    s = jnp.sign(x)
    z = jnp.abs(x)
    t = 1.0 / (1.0 + 0.3275911 * z)
    poly = ((((1.061405429 * t - 1.453152027) * t + 1.421413741) * t
             - 0.284496736) * t + 0.254829592) * t
    return s * (1.0 - poly * jnp.exp(-z * z))


def _gelu(x):
    return x * 0.5 * (1.0 + _erf(x * 0.7071067811865476))


def _norm_act(y, stats, count):
    inv = 1.0 / count
    m = stats[0:1, :] * inv
    v = stats[1:2, :] * inv - m * m
    return _gelu((y - m) * lax.rsqrt(v + EPS))


def _onehot_cats(cv):
    io = lax.broadcasted_iota(jnp.int32, (1, SPAD), 1)
    cvi = cv.astype(jnp.int32)
    parts = [(cvi[:, c:c + 1] == io).astype(jnp.float32) for c in range(CATN)]
    return jnp.concatenate(parts, axis=1)


def _fused_mm(ins, adds, bias, want_stats):
    R = (ins[0][0] if ins else adds[0]).shape[0]
    Ho = ins[0][1].shape[1] if ins else adds[0].shape[1]
    grid = (R // BR,)
    arrays, specs, layout = [], [], []
    for (X, Wt, st, kind) in ins:
        K = X.shape[1]
        arrays.append(X)
        specs.append(pl.BlockSpec((BR, K), lambda i: (i, 0)))
        arrays.append(Wt)
        specs.append(pl.BlockSpec(Wt.shape, lambda i: (0, 0)))
        if st is not None:
            arrays.append(st)
            specs.append(pl.BlockSpec((8, K), lambda i: (0, 0)))
        layout.append(('mm', kind, st is not None))
    for A in adds:
        arrays.append(A)
        specs.append(pl.BlockSpec((BR, Ho), lambda i: (i, 0)))
        layout.append(('add', None, False))
    if bias is not None:
        arrays.append(bias.reshape(1, Ho))
        specs.append(pl.BlockSpec((1, Ho), lambda i: (0, 0)))
    out_shape = [jax.ShapeDtypeStruct((R, Ho), jnp.float32)]
    out_specs = [pl.BlockSpec((BR, Ho), lambda i: (i, 0))]
    if want_stats:
        out_shape.append(jax.ShapeDtypeStruct((8, Ho), jnp.float32))
        out_specs.append(pl.BlockSpec((8, Ho), lambda i: (0, 0)))

    def body(*refs):
        i = pl.program_id(0)
        pos = 0
        acc = None
        for (op, kind, has_st) in layout:
            if op == 'mm':
                xv = refs[pos][...]
                wv = refs[pos + 1][...]
                pos += 2
                if has_st:
                    sv = refs[pos][...]
                    pos += 1
                    xv = _norm_act(xv, sv, float(R))
                if kind == 'onehot':
                    xv = _onehot_cats(xv)
                yv = jnp.dot(xv, wv, preferred_element_type=jnp.float32)
            else:
                yv = refs[pos][...]
                pos += 1
            acc = yv if acc is None else acc + yv
        if bias is not None:
            acc = acc + refs[pos][...]
            pos += 1
        refs[pos][...] = acc
        if want_stats:
            sref = refs[pos + 1]

            @pl.when(i == 0)
            def _():
                sref[...] = jnp.zeros_like(sref)

            sref[0:1, :] += jnp.sum(acc, axis=0, keepdims=True)
            sref[1:2, :] += jnp.sum(acc * acc, axis=0, keepdims=True)

    res = pl.pallas_call(
        body, grid=grid, in_specs=specs, out_specs=out_specs,
        out_shape=out_shape)(*arrays)
    return res if want_stats else res[0]


def _mm2(X, W1t, W2t):
    R, K = X.shape
    Ho = W1t.shape[1]

    def body(x_ref, w1_ref, w2_ref, o1_ref, o2_ref):
        xv = x_ref[...]
        o1_ref[...] = jnp.dot(xv, w1_ref[...], preferred_element_type=jnp.float32)
        o2_ref[...] = jnp.dot(xv, w2_ref[...], preferred_element_type=jnp.float32)

    return pl.pallas_call(
        body, grid=(R // BR,),
        in_specs=[pl.BlockSpec((BR, K), lambda i: (i, 0)),
                  pl.BlockSpec((K, Ho), lambda i: (0, 0)),
                  pl.BlockSpec((K, Ho), lambda i: (0, 0))],
        out_specs=[pl.BlockSpec((BR, Ho), lambda i: (i, 0)),
                   pl.BlockSpec((BR, Ho), lambda i: (i, 0))],
        out_shape=[jax.ShapeDtypeStruct((R, Ho), jnp.float32),
                   jax.ShapeDtypeStruct((R, Ho), jnp.float32)])(X, W1t, W2t)


def _ew_act(ys, base, out_rows=None):
    R = ys[0][0].shape[0]
    Ho = sum(y.shape[1] for (y, _) in ys)
    arrays, specs = [], []
    for (Y, st) in ys:
        K = Y.shape[1]
        arrays.append(Y)
        specs.append(pl.BlockSpec((BR, K), lambda i: (i, 0)))
        arrays.append(st)
        specs.append(pl.BlockSpec((8, K), lambda i: (0, 0)))
    if base is not None:
        arrays.append(base)
        specs.append(pl.BlockSpec((BR, Ho), lambda i: (i, 0)))

    def body(*refs):
        parts = []
        pos = 0
        for _ in ys:
            parts.append(_norm_act(refs[pos][...], refs[pos + 1][...], float(R)))
            pos += 2
        z = parts[0] if len(parts) == 1 else jnp.concatenate(parts, axis=1)
        if base is not None:
            z = refs[pos][...] + z
            pos += 1
        refs[pos][...] = z

    return pl.pallas_call(
        body, grid=(R // BR,), in_specs=specs,
        out_specs=pl.BlockSpec((BR, Ho), lambda i: (i, 0)),
        out_shape=jax.ShapeDtypeStruct((out_rows or R, Ho),
                                       jnp.float32))(*arrays)


def _scale_rows(agg, cnt16):
    Ho = agg.shape[1]
    R = N

    def body(a_ref, c_ref, o_ref):
        c = jnp.clip(c_ref[...][:, 0:1], 1.0, None)
        o_ref[...] = a_ref[...] * (1.0 / c)

    return pl.pallas_call(
        body, grid=(R // BR,),
        in_specs=[pl.BlockSpec((BR, Ho), lambda i: (i, 0)),
                  pl.BlockSpec((BR, CB), lambda i: (i, 0))],
        out_specs=pl.BlockSpec((BR, Ho), lambda i: (i, 0)),
        out_shape=jax.ShapeDtypeStruct((R, Ho), jnp.float32))(agg, cnt16)


def _sc_mesh():
    return plsc.VectorSubcoreMesh(core_axis_name="c", subcore_axis_name="s")


def _gather2_body(tA, iA2, tB, iB2, oA, oB, idxa_v, idxb_v, rows_v, sem):
    wid = lax.axis_index("s") * NC + lax.axis_index("c")
    c0 = wid * WPW
    pltpu.sync_copy(iA2.at[pl.ds(c0, WPW)], idxa_v)
    pltpu.sync_copy(iB2.at[pl.ds(c0, WPW)], idxb_v)

    def one(t, iv, oh):
        def step(k, carry):
            pltpu.async_copy(t.at[iv.at[k]], rows_v, sem).wait()

            @pl.when(c0 + k < NCHUNK)
            def _():
                pltpu.sync_copy(rows_v, oh.at[pl.ds((c0 + k) * EC, EC)])

            return carry

        lax.fori_loop(0, WPW, step, 0)

    one(tA, idxa_v, oA)
    one(tB, idxb_v, oB)


def _sc_gather2(tableA, idxA2, tableB, idxB2):
    return pl.kernel(
        _gather2_body,
        out_type=(jax.ShapeDtypeStruct((E, H), jnp.float32),
                  jax.ShapeDtypeStruct((E, H), jnp.float32)),
        mesh=_sc_mesh(),
        scratch_types=[pltpu.VMEM((WPW, EC), jnp.int32),
                       pltpu.VMEM((WPW, EC), jnp.int32),
                       pltpu.VMEM((EC, H), jnp.float32),
                       pltpu.SemaphoreType.DMA],
    )(tableA, idxA2, tableB, idxB2)


def _scatter_body(vals_h, idx_h, z_h, out_h, idx_v, rows_v, table_sh):
    cid = lax.axis_index("c")
    sid = lax.axis_index("s")
    r0 = sid * RPT
    col0 = cid * CB
    pltpu.sync_copy(z_h, table_sh.at[pl.ds(r0, RPT)])
    plsc.subcore_barrier()

    def step(k, carry):
        c = sid + k * NS

        @pl.when(c < NCHUNK)
        def _():
            base = c * EC
            pltpu.sync_copy(idx_h.at[pl.ds(base, EC)], idx_v)
            pltpu.sync_copy(vals_h.at[pl.ds(base, EC), pl.ds(col0, CB)],
                            rows_v)
            pltpu.sync_copy(rows_v, table_sh.at[idx_v], add=True)

        return carry

    lax.fori_loop(0, CPT, step, 0)
    plsc.subcore_barrier()
    pltpu.sync_copy(table_sh.at[pl.ds(r0, RPT)],
                    out_h.at[pl.ds(r0, RPT), pl.ds(col0, CB)])


def _sc_scatter(vals, idx, z_h):
    return pl.kernel(
        _scatter_body,
        out_type=jax.ShapeDtypeStruct((NPAD, H), jnp.float32),
        mesh=_sc_mesh(),
        scratch_types=[pltpu.VMEM((EC,), jnp.int32),
                       pltpu.VMEM((EC, CB), jnp.float32),
                       pltpu.VMEM_SHARED((NPAD, CB), jnp.float32)],
    )(vals, idx, z_h)


def _count_body(idx_h, z_h, ones_h, out_h, idx_v, ones_v, table_sh):
    cid = lax.axis_index("c")
    sid = lax.axis_index("s")
    r0 = sid * RPT
    pltpu.sync_copy(ones_h, ones_v)
    pltpu.sync_copy(z_h, table_sh.at[pl.ds(r0, RPT)])
    plsc.subcore_barrier()

    def step(k, carry):
        c = sid + k * NS

        @pl.when(c < NCHUNK)
        def _():
            pltpu.sync_copy(idx_h.at[pl.ds(c * EC, EC)], idx_v)
            pltpu.sync_copy(ones_v, table_sh.at[idx_v], add=True)

        return carry

    lax.fori_loop(0, CPT, step, 0)
    plsc.subcore_barrier()

    @pl.when(cid == 0)
    def _():
        pltpu.sync_copy(table_sh.at[pl.ds(r0, RPT)],
                        out_h.at[pl.ds(r0, RPT)])


def _sc_count(idx, z_h, ones_h):
    return pl.kernel(
        _count_body,
        out_type=jax.ShapeDtypeStruct((NPAD, CB), jnp.float32),
        mesh=_sc_mesh(),
        scratch_types=[pltpu.VMEM((EC,), jnp.int32),
                       pltpu.VMEM((EC, CB), jnp.float32),
                       pltpu.VMEM_SHARED((NPAD, CB), jnp.float32)],
    )(idx, z_h, ones_h)


def kernel(x, edge_attr, params, edge_index):
    p = params
    f32 = jnp.float32

    def Wt(name):
        return p[name + '.w'].T

    def Ws(name, lo, hi):
        return p[name + '.w'][:, lo:hi].T

    def b(name):
        return p[name + '.b']

    src = edge_index[0].astype(jnp.int32)
    dst = edge_index[1].astype(jnp.int32)
    dst_g = jnp.pad(dst, (0, EPAD - E)).reshape(NCHUNKP, EC)
    src_g = jnp.pad(src, (0, EPAD - E)).reshape(NCHUNKP, EC)
    num = edge_attr[:, :NUMF]
    catv = edge_attr[:, NUMF:]

    z640 = jnp.zeros((RPT, CB), f32)
    ones128 = jnp.ones((EC, CB), f32)

    t1, s1 = _fused_mm([(x, Wt('node1'), None, 'id')], [], b('node1'), True)
    t2, s2 = _fused_mm([(t1, Wt('node2'), s1, 'act')], [], b('node2'), True)
    node = _ew_act([(t2, s2)], None)

    u1, su1 = _fused_mm([(num, Wt('num1'), None, 'id')], [], b('num1'), True)
    u2, su2 = _fused_mm([(u1, Wt('num2'), su1, 'act')], [], b('num2'), True)

    Wemb = jnp.zeros((CATN * SPAD, CATN * CAT_DIM), f32)
    for i in range(CATN):
        Wemb = Wemb.at[i * SPAD:i * SPAD + CAT_SIZES[i],
                       i * CAT_DIM:(i + 1) * CAT_DIM].set(p['emb%d' % i])
    ecr, sec = _fused_mm([(catv, Wemb, None, 'onehot')], [], None, True)
    c1, sc1 = _fused_mm([(ecr, Wt('cat1'), sec, 'act')], [], b('cat1'), True)
    edge = _ew_act([(u2, su2), (c1, sc1)], None)

    cnt16 = _sc_count(dst, z640, ones128)

    for l in range(3):
        nm, um, em = 'msg%d' % l, 'upd%d' % l, 'edg%d' % l
        Pd, Ps = _mm2(node, Ws(nm, 0, H), Ws(nm, H, 2 * H))
        Gd, Gs = _sc_gather2(Pd, dst_g, Ps, src_g)
        m_raw, sm = _fused_mm([(edge, Ws(nm, 2 * H, 3 * H), None, 'id')],
                              [Gd, Gs], b(nm), True)
        msg = _ew_act([(m_raw, sm)], None)
        aggs = _sc_scatter(msg, dst, z640)
        aggm = _scale_rows(aggs, cnt16)
        u_raw, su = _fused_mm([(node, Ws(um, 0, H), None, 'id'),
                               (aggm, Ws(um, H, 2 * H), None, 'id')],
                              [], b(um), True)
        node = _ew_act([(u_raw, su)], node)
        Qd, Qs = _mm2(node, Ws(em, H, 2 * H), Ws(em, 2 * H, 3 * H))
        Hd, Hs = _sc_gather2(Qd, dst_g, Qs, src_g)
        e_raw, se = _fused_mm([(edge, Ws(em, 0, H), None, 'id')],
                              [Hd, Hs], b(em), True)
        edge = _ew_act([(e_raw, se)], edge)

    Rs, Rd = _mm2(node, Ws('fin1', 0, H), Ws('fin1', H, 2 * H))
    Fs, Fd = _sc_gather2(Rs, src_g, Rd, dst_g)
    g_raw, sg = _fused_mm([(edge, Ws('fin1', 2 * H, 3 * H), None, 'id')],
                          [Fs, Fd], b('fin1'), True)
    out = _fused_mm([(g_raw, Wt('fin2'), sg, 'act')], [], b('fin2'), False)
    return out


# speedup vs baseline: 1.3780x; 1.3780x over previous
import jax
import jax.numpy as jnp
from jax import lax
from jax.experimental import pallas as pl
from jax.experimental.pallas import tpu as pltpu
from jax.experimental.pallas import tpu_sc as plsc

N = 10000
E = 160000
H = 256
NUMF = 16
CATN = 4
CAT_SIZES = (8, 12, 16, 24)
SPAD = 32
CAT_DIM = 16
EPS = 1e-5

BR = 5000

NC = 2
NS = 16
NW = NC * NS
EC = 128
NCHUNK = -(-E // EC)
WPW = -(-NCHUNK // NW)
NCHUNKP = WPW * NW
EPAD = NCHUNKP * EC
CPT = -(-NCHUNK // NS)
NPAD = 10240
RPT = NPAD // NS
CB = 128
NBUF = 3


def _erf(x):
    s = jnp.sign(x)
    z = jnp.abs(x)
    t = 1.0 / (1.0 + 0.3275911 * z)
    poly = ((((1.061405429 * t - 1.453152027) * t + 1.421413741) * t
             - 0.284496736) * t + 0.254829592) * t
    return s * (1.0 - poly * jnp.exp(-z * z))


def _gelu(x):
    return x * 0.5 * (1.0 + _erf(x * 0.7071067811865476))


def _norm_act(y, stats, count):
    inv = 1.0 / count
    m = stats[0:1, :] * inv
    v = stats[1:2, :] * inv - m * m
    return _gelu((y - m) * lax.rsqrt(v + EPS))


def _onehot_cats(cv):
    io = lax.broadcasted_iota(jnp.int32, (1, SPAD), 1)
    cvi = cv.astype(jnp.int32)
    parts = [(cvi[:, c:c + 1] == io).astype(jnp.float32) for c in range(CATN)]
    return jnp.concatenate(parts, axis=1)


def _fused_mm(ins, adds, bias, want_stats):
    R = (ins[0][0] if ins else adds[0]).shape[0]
    Ho = ins[0][1].shape[1] if ins else adds[0].shape[1]
    grid = (R // BR,)
    arrays, specs, layout = [], [], []
    for (X, Wt, st, kind) in ins:
        K = X.shape[1]
        arrays.append(X)
        specs.append(pl.BlockSpec((BR, K), lambda i: (i, 0)))
        arrays.append(Wt)
        specs.append(pl.BlockSpec(Wt.shape, lambda i: (0, 0)))
        if st is not None:
            arrays.append(st)
            specs.append(pl.BlockSpec((8, K), lambda i: (0, 0)))
        layout.append(('mm', kind, st is not None))
    for A in adds:
        arrays.append(A)
        specs.append(pl.BlockSpec((BR, Ho), lambda i: (i, 0)))
        layout.append(('add', None, False))
    if bias is not None:
        arrays.append(bias.reshape(1, Ho))
        specs.append(pl.BlockSpec((1, Ho), lambda i: (0, 0)))
    out_shape = [jax.ShapeDtypeStruct((R, Ho), jnp.float32)]
    out_specs = [pl.BlockSpec((BR, Ho), lambda i: (i, 0))]
    if want_stats:
        out_shape.append(jax.ShapeDtypeStruct((8, Ho), jnp.float32))
        out_specs.append(pl.BlockSpec((8, Ho), lambda i: (0, 0)))

    def body(*refs):
        i = pl.program_id(0)
        pos = 0
        acc = None
        for (op, kind, has_st) in layout:
            if op == 'mm':
                xv = refs[pos][...]
                wv = refs[pos + 1][...]
                pos += 2
                if has_st:
                    sv = refs[pos][...]
                    pos += 1
                    xv = _norm_act(xv, sv, float(R))
                if kind == 'onehot':
                    xv = _onehot_cats(xv)
                yv = jnp.dot(xv, wv, preferred_element_type=jnp.float32)
            else:
                yv = refs[pos][...]
                pos += 1
            acc = yv if acc is None else acc + yv
        if bias is not None:
            acc = acc + refs[pos][...]
            pos += 1
        refs[pos][...] = acc
        if want_stats:
            sref = refs[pos + 1]

            @pl.when(i == 0)
            def _():
                sref[...] = jnp.zeros_like(sref)

            sref[0:1, :] += jnp.sum(acc, axis=0, keepdims=True)
            sref[1:2, :] += jnp.sum(acc * acc, axis=0, keepdims=True)

    res = pl.pallas_call(
        body, grid=grid, in_specs=specs, out_specs=out_specs,
        out_shape=out_shape)(*arrays)
    return res if want_stats else res[0]


def _mm2(X, W1t, W2t):
    R, K = X.shape
    Ho = W1t.shape[1]

    def body(x_ref, w1_ref, w2_ref, o1_ref, o2_ref):
        xv = x_ref[...]
        o1_ref[...] = jnp.dot(xv, w1_ref[...], preferred_element_type=jnp.float32)
        o2_ref[...] = jnp.dot(xv, w2_ref[...], preferred_element_type=jnp.float32)

    return pl.pallas_call(
        body, grid=(R // BR,),
        in_specs=[pl.BlockSpec((BR, K), lambda i: (i, 0)),
                  pl.BlockSpec((K, Ho), lambda i: (0, 0)),
                  pl.BlockSpec((K, Ho), lambda i: (0, 0))],
        out_specs=[pl.BlockSpec((BR, Ho), lambda i: (i, 0)),
                   pl.BlockSpec((BR, Ho), lambda i: (i, 0))],
        out_shape=[jax.ShapeDtypeStruct((R, Ho), jnp.float32),
                   jax.ShapeDtypeStruct((R, Ho), jnp.float32)])(X, W1t, W2t)


def _ew_act(ys, base, out_rows=None):
    R = ys[0][0].shape[0]
    Ho = sum(y.shape[1] for (y, _) in ys)
    arrays, specs = [], []
    for (Y, st) in ys:
        K = Y.shape[1]
        arrays.append(Y)
        specs.append(pl.BlockSpec((BR, K), lambda i: (i, 0)))
        arrays.append(st)
        specs.append(pl.BlockSpec((8, K), lambda i: (0, 0)))
    if base is not None:
        arrays.append(base)
        specs.append(pl.BlockSpec((BR, Ho), lambda i: (i, 0)))

    def body(*refs):
        parts = []
        pos = 0
        for _ in ys:
            parts.append(_norm_act(refs[pos][...], refs[pos + 1][...], float(R)))
            pos += 2
        z = parts[0] if len(parts) == 1 else jnp.concatenate(parts, axis=1)
        if base is not None:
            z = refs[pos][...] + z
            pos += 1
        refs[pos][...] = z

    return pl.pallas_call(
        body, grid=(R // BR,), in_specs=specs,
        out_specs=pl.BlockSpec((BR, Ho), lambda i: (i, 0)),
        out_shape=jax.ShapeDtypeStruct((out_rows or R, Ho),
                                       jnp.float32))(*arrays)


def _scale_rows(agg, cnt16):
    Ho = agg.shape[1]
    R = N

    def body(a_ref, c_ref, o_ref):
        c = jnp.clip(c_ref[...][:, 0:1], 1.0, None)
        o_ref[...] = a_ref[...] * (1.0 / c)

    return pl.pallas_call(
        body, grid=(R // BR,),
        in_specs=[pl.BlockSpec((BR, Ho), lambda i: (i, 0)),
                  pl.BlockSpec((BR, CB), lambda i: (i, 0))],
        out_specs=pl.BlockSpec((BR, Ho), lambda i: (i, 0)),
        out_shape=jax.ShapeDtypeStruct((R, Ho), jnp.float32))(agg, cnt16)


def _sc_mesh():
    return plsc.VectorSubcoreMesh(core_axis_name="c", subcore_axis_name="s")


def _gather2_body(tA, iA, tB, iB, oA, oB, idx_v, rows_v, sem):
    wid = lax.axis_index("s") * NC + lax.axis_index("c")

    def one(t, ih, oh):
        def step(k, carry):
            c = wid + k * NW

            @pl.when(c < NCHUNK)
            def _():
                base = c * EC
                pltpu.sync_copy(ih.at[pl.ds(base, EC)], idx_v)
                pltpu.async_copy(t.at[idx_v], rows_v, sem).wait()
                pltpu.sync_copy(rows_v, oh.at[pl.ds(base, EC)])

            return carry

        lax.fori_loop(0, WPW, step, 0)

    one(tA, iA, oA)
    one(tB, iB, oB)


def _sc_gather2(tableA, idxA, tableB, idxB):
    return pl.kernel(
        _gather2_body,
        out_type=(jax.ShapeDtypeStruct((E, H), jnp.float32),
                  jax.ShapeDtypeStruct((E, H), jnp.float32)),
        mesh=_sc_mesh(),
        scratch_types=[pltpu.VMEM((EC,), jnp.int32),
                       pltpu.VMEM((EC, H), jnp.float32),
                       pltpu.SemaphoreType.DMA],
    )(tableA, idxA, tableB, idxB)


def _scatter_body(vals_h, idx_h, z_h, out_h, idx_v, rows_v, table_sh):
    cid = lax.axis_index("c")
    sid = lax.axis_index("s")
    r0 = sid * RPT
    col0 = cid * CB
    pltpu.sync_copy(z_h, table_sh.at[pl.ds(r0, RPT)])
    plsc.subcore_barrier()

    def step(k, carry):
        c = sid + k * NS

        @pl.when(c < NCHUNK)
        def _():
            base = c * EC
            pltpu.sync_copy(idx_h.at[pl.ds(base, EC)], idx_v)
            pltpu.sync_copy(vals_h.at[pl.ds(base, EC), pl.ds(col0, CB)],
                            rows_v)
            pltpu.sync_copy(rows_v, table_sh.at[idx_v], add=True)

        return carry

    lax.fori_loop(0, CPT, step, 0)
    plsc.subcore_barrier()
    pltpu.sync_copy(table_sh.at[pl.ds(r0, RPT)],
                    out_h.at[pl.ds(r0, RPT), pl.ds(col0, CB)])


def _sc_scatter(vals, idx, z_h):
    return pl.kernel(
        _scatter_body,
        out_type=jax.ShapeDtypeStruct((NPAD, H), jnp.float32),
        mesh=_sc_mesh(),
        scratch_types=[pltpu.VMEM((EC,), jnp.int32),
                       pltpu.VMEM((EC, CB), jnp.float32),
                       pltpu.VMEM_SHARED((NPAD, CB), jnp.float32)],
    )(vals, idx, z_h)


def _count_body(idx_h, z_h, ones_h, out_h, idx_v, ones_v, table_sh):
    cid = lax.axis_index("c")
    sid = lax.axis_index("s")
    r0 = sid * RPT
    pltpu.sync_copy(ones_h, ones_v)
    pltpu.sync_copy(z_h, table_sh.at[pl.ds(r0, RPT)])
    plsc.subcore_barrier()

    def step(k, carry):
        c = sid + k * NS

        @pl.when(c < NCHUNK)
        def _():
            pltpu.sync_copy(idx_h.at[pl.ds(c * EC, EC)], idx_v)
            pltpu.sync_copy(ones_v, table_sh.at[idx_v], add=True)

        return carry

    lax.fori_loop(0, CPT, step, 0)
    plsc.subcore_barrier()

    @pl.when(cid == 0)
    def _():
        pltpu.sync_copy(table_sh.at[pl.ds(r0, RPT)],
                        out_h.at[pl.ds(r0, RPT)])


def _sc_count(idx, z_h, ones_h):
    return pl.kernel(
        _count_body,
        out_type=jax.ShapeDtypeStruct((NPAD, CB), jnp.float32),
        mesh=_sc_mesh(),
        scratch_types=[pltpu.VMEM((EC,), jnp.int32),
                       pltpu.VMEM((EC, CB), jnp.float32),
                       pltpu.VMEM_SHARED((NPAD, CB), jnp.float32)],
    )(idx, z_h, ones_h)


def kernel(x, edge_attr, params, edge_index):
    p = params
    f32 = jnp.float32

    def Wt(name):
        return p[name + '.w'].T

    def Ws(name, lo, hi):
        return p[name + '.w'][:, lo:hi].T

    def b(name):
        return p[name + '.b']

    src = edge_index[0].astype(jnp.int32)
    dst = edge_index[1].astype(jnp.int32)
    num = edge_attr[:, :NUMF]
    catv = edge_attr[:, NUMF:]

    z640 = jnp.zeros((RPT, CB), f32)
    ones128 = jnp.ones((EC, CB), f32)

    t1, s1 = _fused_mm([(x, Wt('node1'), None, 'id')], [], b('node1'), True)
    t2, s2 = _fused_mm([(t1, Wt('node2'), s1, 'act')], [], b('node2'), True)
    node = _ew_act([(t2, s2)], None)

    u1, su1 = _fused_mm([(num, Wt('num1'), None, 'id')], [], b('num1'), True)
    u2, su2 = _fused_mm([(u1, Wt('num2'), su1, 'act')], [], b('num2'), True)

    Wemb = jnp.zeros((CATN * SPAD, CATN * CAT_DIM), f32)
    for i in range(CATN):
        Wemb = Wemb.at[i * SPAD:i * SPAD + CAT_SIZES[i],
                       i * CAT_DIM:(i + 1) * CAT_DIM].set(p['emb%d' % i])
    ecr, sec = _fused_mm([(catv, Wemb, None, 'onehot')], [], None, True)
    c1, sc1 = _fused_mm([(ecr, Wt('cat1'), sec, 'act')], [], b('cat1'), True)
    edge = _ew_act([(u2, su2), (c1, sc1)], None)

    cnt16 = _sc_count(dst, z640, ones128)

    for l in range(3):
        nm, um, em = 'msg%d' % l, 'upd%d' % l, 'edg%d' % l
        Pd, Ps = _mm2(node, Ws(nm, 0, H), Ws(nm, H, 2 * H))
        Gd, Gs = _sc_gather2(Pd, dst, Ps, src)
        m_raw, sm = _fused_mm([(edge, Ws(nm, 2 * H, 3 * H), None, 'id')],
                              [Gd, Gs], b(nm), True)
        msg = _ew_act([(m_raw, sm)], None)
        aggs = _sc_scatter(msg, dst, z640)
        aggm = _scale_rows(aggs, cnt16)
        u_raw, su = _fused_mm([(node, Ws(um, 0, H), None, 'id'),
                               (aggm, Ws(um, H, 2 * H), None, 'id')],
                              [], b(um), True)
        node = _ew_act([(u_raw, su)], node)
        Qd, Qs = _mm2(node, Ws(em, H, 2 * H), Ws(em, 2 * H, 3 * H))
        Hd, Hs = _sc_gather2(Qd, dst, Qs, src)
        e_raw, se = _fused_mm([(edge, Ws(em, 0, H), None, 'id')],
                              [Hd, Hs], b(em), True)
        edge = _ew_act([(e_raw, se)], edge)

    Rs, Rd = _mm2(node, Ws('fin1', 0, H), Ws('fin1', H, 2 * H))
    Fs, Fd = _sc_gather2(Rs, src, Rd, dst)
    g_raw, sg = _fused_mm([(edge, Ws('fin1', 2 * H, 3 * H), None, 'id')],
                          [Fs, Fd], b('fin1'), True)
    out = _fused_mm([(g_raw, Wt('fin2'), sg, 'act')], [], b('fin2'), False)
    return out
